# baseline (speedup 1.0000x reference)
import functools

import jax
import jax.numpy as jnp
import numpy as np
from jax import lax
from jax.experimental import pallas as pl
from jax.experimental.pallas import tpu as pltpu
from jax.experimental.pallas import tpu_sc as plsc

XMIN = 0.0
XMAX = 1.0

NC = 2
NS = 16
L = 16
NW = NC * NS

D = 16
C = 2048
GROUP = 128
G = C // GROUP
JJ = C // L

CBR = 496
BCH = 8
RB = CBR * BCH
INW = 8 * CBR + 16


def _build_body(cpad8_hbm, c2x_hbm, out_v, sem):
    wid = lax.axis_index("s") * NC + lax.axis_index("c")
    rbase = wid * RB

    def chunk_body(cb, carry):
        roff = rbase + cb * CBR
        pltpu.sync_copy(cpad8_hbm.at[pl.ds(roff, CBR)],
                        out_v.at[:, pl.ds(0, 8)])
        pltpu.sync_copy(cpad8_hbm.at[pl.ds(roff + 1, CBR)],
                        out_v.at[:, pl.ds(8, 8)])
        pltpu.sync_copy(out_v, c2x_hbm.at[pl.ds(roff, CBR)])
        return carry

    lax.fori_loop(0, BCH, chunk_body, 0)


def _spline_body(n, nbr, pt, x_hbm, tbl_hbm, y_hbm, *refs):
    (x_a, q_a, pos_a, w0_a, w1_a, w2_a, w3_a, rows_a, y_a,
     x_b, q_b, pos_b, w0_b, w1_b, w2_b, w3_b, rows_b, y_b,
     sem_a, sem_b) = refs
    sets = (
        (x_a, q_a, pos_a, (w0_a, w1_a, w2_a, w3_a), rows_a, y_a, sem_a),
        (x_b, q_b, pos_b, (w0_b, w1_b, w2_b, w3_b), rows_b, y_b, sem_b),
    )
    inv_h = (n - 1) / (XMAX - XMIN)
    sixth = 1.0 / 6.0
    eps = float(np.finfo(np.float32).eps)
    wid = lax.axis_index("s") * NC + lax.axis_index("c")
    base = wid * pt
    nch = pt // C

    def stage1(g, s):
        x_v, q_v, pos_v, wv4, rows_v, _, sem = sets[s]
        off = base + g * C
        pltpu.sync_copy(x_hbm.at[pl.ds(off, C)], x_v)

        def p1(j, carry):
            sl = pl.ds(j * L, L)
            xv = x_v[sl]
            t = (xv - XMIN) * inv_h
            ii = t.astype(jnp.int32)
            u = jnp.minimum(jnp.maximum(t - ii.astype(jnp.float32), 0.0),
                            1.0 - eps)
            u2 = u * u
            u3 = u2 * u
            sm = 1.0 - u
            wv4[0][sl] = sm * sm * sm * sixth
            wv4[1][sl] = (3.0 * u3 - 6.0 * u2 + 4.0) * sixth
            wv4[2][sl] = (-3.0 * u3 + 3.0 * u2 + 3.0 * u + 1.0) * sixth
            wv4[3][sl] = u3 * sixth
            ii = jnp.minimum(jnp.maximum(ii, 0), n - 1)
            q = jnp.minimum(jnp.maximum((ii - 1) >> 3, 0), nbr - 1)
            qd = q * 8
            cl0 = jnp.maximum(ii - 1, 0)
            cl3 = jnp.minimum(ii + 2, n - 1)
            p0 = jnp.clip(cl0 - qd, 0, D - 1)
            p1_ = jnp.clip(ii - qd, 0, D - 1)
            p2_ = jnp.clip(ii + 1 - qd, 0, D - 1)
            p3 = jnp.clip(cl3 - qd, 0, D - 1)
            pos_v[sl] = p0 | (p1_ << 4) | (p2_ << 8) | (p3 << 12)
            q_v[sl] = q
            return carry

        lax.fori_loop(0, JJ, p1, 0)
        for b in range(G):
            gsl = pl.ds(b * GROUP, GROUP)
            pltpu.async_copy(tbl_hbm.at[q_v.at[gsl]], rows_v.at[gsl], sem)

    def stage2(g, s):
        _, _, pos_v, wv4, rows_v, y_v, sem = sets[s]
        off = base + g * C
        pltpu.make_async_copy(tbl_hbm.at[pl.ds(0, C)], rows_v, sem).wait()

        def p2(j, carry):
            sl = pl.ds(j * L, L)
            pv = j * L + lax.iota(jnp.int32, L)
            packed = pos_v[sl]
            acc = None
            for k in range(4):
                pk = (packed >> (4 * k)) & 15
                ck = plsc.load_gather(rows_v, [pv, pk])
                wk = wv4[k][sl]
                acc = wk * ck if acc is None else acc + wk * ck
            y_v[sl] = acc
            return carry

        lax.fori_loop(0, JJ, p2, 0)
        pltpu.sync_copy(y_v, y_hbm.at[pl.ds(off, C)])

    stage1(0, 0)

    def pair(g2, carry):
        ga = 2 * g2
        gb = ga + 1
        stage1(gb, 1)
        stage2(ga, 0)

        @pl.when(g2 + 1 < nch // 2)
        def _():
            stage1(ga + 2, 0)

        stage2(gb, 1)
        return carry

    lax.fori_loop(0, nch // 2, pair, 0)


def kernel(x, coeff):
    n = coeff.shape[0]
    shape = x.shape
    xf = x.reshape(-1)
    p = xf.shape[0]
    per_tile = 2 * C
    tile_pts = -(-p // (NW * per_tile)) * per_tile
    p_pad = tile_pts * NW
    if p_pad != p:
        xf = jnp.pad(xf, (0, p_pad - p))

    nbr = NW * RB
    cpad_len = 8 * CBR * BCH * NW + INW - 8 * CBR
    cpad = jnp.pad(coeff, (0, cpad_len - n), mode="edge")
    cpad8 = jax.lax.optimization_barrier(cpad.reshape(-1, 8))

    mesh = plsc.VectorSubcoreMesh(core_axis_name="c", subcore_axis_name="s")
    params = pltpu.CompilerParams(
        use_tc_tiling_on_sc=False, needs_layout_passes=False)

    build = pl.kernel(
        _build_body,
        out_type=jax.ShapeDtypeStruct((nbr, D), jnp.float32),
        mesh=mesh,
        compiler_params=params,
        scratch_types=[
            pltpu.VMEM((CBR, D), jnp.float32),
            pltpu.SemaphoreType.DMA,
        ],
    )
    tbl = build(cpad8)

    buf_set = [
        pltpu.VMEM((C,), jnp.float32),
        pltpu.VMEM((C,), jnp.int32),
        pltpu.VMEM((C,), jnp.int32),
        pltpu.VMEM((C,), jnp.float32),
        pltpu.VMEM((C,), jnp.float32),
        pltpu.VMEM((C,), jnp.float32),
        pltpu.VMEM((C,), jnp.float32),
        pltpu.VMEM((C, D), jnp.float32),
        pltpu.VMEM((C,), jnp.float32),
    ]
    run = pl.kernel(
        functools.partial(_spline_body, n, nbr, tile_pts),
        out_type=jax.ShapeDtypeStruct((p_pad,), jnp.float32),
        mesh=mesh,
        compiler_params=params,
        scratch_types=buf_set + buf_set
        + [pltpu.SemaphoreType.DMA, pltpu.SemaphoreType.DMA],
    )
    y = run(xf, tbl)
    if p_pad != p:
        y = y[:p]
    return y.reshape(shape)
